# Initial kernel scaffold; baseline (speedup 1.0000x reference)
import jax
import jax.numpy as jnp
from jax.experimental import pallas as pl


def kernel(logits):
    raise NotImplementedError("write your pallas kernel here")


# speedup vs baseline: 131.3222x; 131.3222x over previous
import math

import jax
import jax.numpy as jnp
from jax.experimental import pallas as pl
from jax.experimental.pallas import tpu as pltpu

_K = 8
_BSZ = 128
_N = 1000
_ROWS = 16


def _expm1(x):
    u = jnp.exp(x)
    um1 = u - 1.0
    return jnp.where(u == 1.0, x,
                     jnp.where(um1 == -1.0, -1.0, um1 * x / jnp.log(u)))


def _log1mexp(x):
    mask = (-math.log(2.0)) < x
    return jnp.where(mask, jnp.log(-_expm1(x)), jnp.log1p(-jnp.exp(x)))


def _logaddexp_c(x1, x2):
    delta = jnp.where(x1 == x2, 0.0, x1 - x2)
    return jnp.maximum(x1, x2) + jax.nn.softplus(-jnp.abs(delta))


def _body(logits_t_ref, u_ref, out_ref, a_ref, lp_ref):
    neg_inf = jnp.float32(-jnp.inf)

    lp = jnp.minimum(jax.nn.log_sigmoid(logits_t_ref[...]), -1e-07)
    lq = _log1mexp(lp)
    lp_ref[...] = lp

    rows = jax.lax.broadcasted_iota(jnp.int32, (_ROWS, _BSZ), 0)
    state0 = jnp.where(rows == 1, 0.0, neg_inf)
    a_ref[0] = state0

    def dp_step(t, state):
        lp_row = lp_ref[pl.ds(t, 1), :]
        lq_row = _log1mexp(lp_row)
        shifted = jnp.concatenate(
            [jnp.full((1, _BSZ), neg_inf, jnp.float32), state[:-1, :]], axis=0)
        new = _logaddexp_c(shifted + lp_row, state + lq_row)
        a_ref[pl.ds(t + 1, 1)] = new[None]
        return new

    jax.lax.fori_loop(0, _N, dp_step, state0)

    del lq

    def s_step(t, j):
        i = _N - t
        slab_i = a_ref[pl.ds(i, 1)][0]
        slab_im1 = a_ref[pl.ds(i - 1, 1)][0]
        z = jnp.max(jnp.where(rows == j, slab_i, neg_inf), axis=0, keepdims=True)
        pg = jnp.max(jnp.where(rows == (j - 1), slab_im1, neg_inf), axis=0,
                     keepdims=True)
        lp_row = lp_ref[pl.ds(i - 1, 1), :]
        p = pg + lp_row - z
        q = _log1mexp(p)
        u = u_ref[pl.ds(t, 1), :]
        x = (u < jax.nn.sigmoid(p - q)).astype(jnp.float32)
        out_ref[pl.ds(i - 1, 1), :] = x
        return jnp.where(x > 0, j - 1, j)

    j0 = jnp.full((1, _BSZ), _K + 1, jnp.int32)
    jax.lax.fori_loop(0, _N, s_step, j0)


def _uniforms():
    keys = jax.random.split(jax.random.key(42), _N)
    return jax.vmap(lambda k: jax.random.uniform(k, (_BSZ,)))(keys)


def kernel(logits):
    us = _uniforms()
    out_t = pl.pallas_call(
        _body,
        out_shape=jax.ShapeDtypeStruct((_N, _BSZ), jnp.float32),
        in_specs=[
            pl.BlockSpec(memory_space=pltpu.VMEM),
            pl.BlockSpec(memory_space=pltpu.VMEM),
        ],
        out_specs=pl.BlockSpec(memory_space=pltpu.VMEM),
        scratch_shapes=[
            pltpu.VMEM((_N + 1, _ROWS, _BSZ), jnp.float32),
            pltpu.VMEM((_N, _BSZ), jnp.float32),
        ],
    )(logits.T, us)
    return out_t.T


# speedup vs baseline: 168.8395x; 1.2857x over previous
import math

import jax
import jax.numpy as jnp
from jax.experimental import pallas as pl
from jax.experimental.pallas import tpu as pltpu

_K = 8
_BSZ = 128
_N = 1000
_ROWS = 16


def _expm1(x):
    u = jnp.exp(x)
    um1 = u - 1.0
    return jnp.where(u == 1.0, x,
                     jnp.where(um1 == -1.0, -1.0, um1 * x / jnp.log(u)))


def _log1mexp(x):
    mask = (-math.log(2.0)) < x
    return jnp.where(mask, jnp.log(-_expm1(x)), jnp.log1p(-jnp.exp(x)))


def _logaddexp_c(x1, x2):
    delta = jnp.where(x1 == x2, 0.0, x1 - x2)
    return jnp.maximum(x1, x2) + jax.nn.softplus(-jnp.abs(delta))


def _body(logits_t_ref, u_ref, out_ref, lp_ref, lq_ref, d_ref):
    neg_inf = jnp.float32(-jnp.inf)

    lp = jnp.minimum(jax.nn.log_sigmoid(logits_t_ref[...]), -1e-07)
    lp_ref[...] = lp
    lq_ref[...] = _log1mexp(lp)

    rows = jax.lax.broadcasted_iota(jnp.int32, (_ROWS, _BSZ), 0)
    rows_valid = (rows >= 1) & (rows <= _K + 1)
    state0 = jnp.where(rows == 1, 0.0, neg_inf)

    def dp_step(t, state):
        lp_row = lp_ref[pl.ds(t, 1), :]
        lq_row = lq_ref[pl.ds(t, 1), :]
        s_lo = jnp.concatenate(
            [jnp.full((1, _BSZ), neg_inf, jnp.float32), state[:-1, :]],
            axis=0) + lp_row
        new = _logaddexp_c(s_lo, state + lq_row)
        p = s_lo - new
        q = _log1mexp(p)
        u_row = u_ref[pl.ds(_N - 1 - t, 1), :]
        bit = (u_row < jax.nn.sigmoid(p - q)).astype(jnp.int32)
        word = jnp.sum(jnp.where(rows_valid, bit << rows, 0), axis=0,
                       keepdims=True)
        d_ref[pl.ds(t, 1), :] = word
        return new

    jax.lax.fori_loop(0, _N, dp_step, state0)

    def s_step(t, j):
        w = d_ref[pl.ds(_N - 1 - t, 1), :]
        bit = (w >> j) & 1
        out_ref[pl.ds(_N - 1 - t, 1), :] = bit.astype(jnp.float32)
        return j - bit

    j0 = jnp.full((1, _BSZ), _K + 1, jnp.int32)
    jax.lax.fori_loop(0, _N, s_step, j0)


def _uniforms():
    keys = jax.random.split(jax.random.key(42), _N)
    return jax.vmap(lambda k: jax.random.uniform(k, (_BSZ,)))(keys)


def kernel(logits):
    us = _uniforms()
    out_t = pl.pallas_call(
        _body,
        out_shape=jax.ShapeDtypeStruct((_N, _BSZ), jnp.float32),
        in_specs=[
            pl.BlockSpec(memory_space=pltpu.VMEM),
            pl.BlockSpec(memory_space=pltpu.VMEM),
        ],
        out_specs=pl.BlockSpec(memory_space=pltpu.VMEM),
        scratch_shapes=[
            pltpu.VMEM((_N, _BSZ), jnp.float32),
            pltpu.VMEM((_N, _BSZ), jnp.float32),
            pltpu.VMEM((_N, _BSZ), jnp.int32),
        ],
    )(logits.T, us)
    return out_t.T


# speedup vs baseline: 275.1805x; 1.6298x over previous
import math

import jax
import jax.numpy as jnp
from jax.experimental import pallas as pl
from jax.experimental.pallas import tpu as pltpu

_K = 8
_BSZ = 128
_N = 1000
_ROWS = 16


def _expm1(x):
    u = jnp.exp(x)
    um1 = u - 1.0
    return jnp.where(u == 1.0, x,
                     jnp.where(um1 == -1.0, -1.0, um1 * x / jnp.log(u)))


def _log1mexp(x):
    mask = (-math.log(2.0)) < x
    return jnp.where(mask, jnp.log(-_expm1(x)), jnp.log1p(-jnp.exp(x)))


def _logaddexp_c(x1, x2):
    delta = jnp.where(x1 == x2, 0.0, x1 - x2)
    return jnp.maximum(x1, x2) + jax.nn.softplus(-jnp.abs(delta))


def _body(logits_t_ref, u_ref, out_ref, lp_ref, lq_ref, d_ref):
    neg_inf = jnp.float32(-jnp.inf)

    lp = jnp.minimum(jax.nn.log_sigmoid(logits_t_ref[...]), -1e-07)
    lp_ref[...] = lp
    lq_ref[...] = _log1mexp(lp)

    rows = jax.lax.broadcasted_iota(jnp.int32, (_ROWS, _BSZ), 0)
    rows_valid = (rows >= 1) & (rows <= _K + 1)
    state0 = jnp.where(rows == 1, 0.0, neg_inf)

    def dp_step(t, state):
        lp_row = lp_ref[pl.ds(t, 1), :]
        lq_row = lq_ref[pl.ds(t, 1), :]
        s_lo = jnp.concatenate(
            [jnp.full((1, _BSZ), neg_inf, jnp.float32), state[:-1, :]],
            axis=0) + lp_row
        new = _logaddexp_c(s_lo, state + lq_row)
        p = s_lo - new
        u_row = u_ref[pl.ds(_N - 1 - t, 1), :]
        bit = (u_row < jnp.exp(p)).astype(jnp.int32)
        word = jnp.sum(jnp.where(rows_valid, bit << rows, 0), axis=0,
                       keepdims=True)
        d_ref[pl.ds(t, 1), :] = word
        return new

    jax.lax.fori_loop(0, _N, dp_step, state0)

    def s_step(t, j):
        w = d_ref[pl.ds(_N - 1 - t, 1), :]
        bit = (w >> j) & 1
        out_ref[pl.ds(_N - 1 - t, 1), :] = bit.astype(jnp.float32)
        return j - bit

    j0 = jnp.full((1, _BSZ), _K + 1, jnp.int32)
    jax.lax.fori_loop(0, _N, s_step, j0)


def _uniforms():
    keys = jax.random.split(jax.random.key(42), _N)
    return jax.vmap(lambda k: jax.random.uniform(k, (_BSZ,)))(keys)


def kernel(logits):
    us = _uniforms()
    out_t = pl.pallas_call(
        _body,
        out_shape=jax.ShapeDtypeStruct((_N, _BSZ), jnp.float32),
        in_specs=[
            pl.BlockSpec(memory_space=pltpu.VMEM),
            pl.BlockSpec(memory_space=pltpu.VMEM),
        ],
        out_specs=pl.BlockSpec(memory_space=pltpu.VMEM),
        scratch_shapes=[
            pltpu.VMEM((_N, _BSZ), jnp.float32),
            pltpu.VMEM((_N, _BSZ), jnp.float32),
            pltpu.VMEM((_N, _BSZ), jnp.int32),
        ],
    )(logits.T, us)
    return out_t.T
